# baseline (speedup 1.0000x reference)
import functools

import jax
import jax.numpy as jnp
from jax import lax
from jax.experimental import pallas as pl
from jax.experimental.pallas import tpu as pltpu
from jax.experimental.pallas import tpu_sc as plsc

DIM = 64
BATCH = 16384
NENT = 1000000
NTAB = 1000
_L = 16
_NC = 2
_NS = 16
_NW = _NC * _NS
_BPW = BATCH // _NW
_CH = 32
_NCHUNK = _BPW // _CH

_RANGE = 31232
_SLAB = 512
_TAIL0 = _RANGE * 31 + 31744
_NTAIL = NENT - _TAIL0
_ICH = 1024
_HCAP = 2 * BATCH + 4 * _L
_SCAP = 4096
_UCAP = 4096 + 4 * _L
_SO_ROWS = 2 * BATCH + _L
_DUMP = 2 * BATCH


def _proj_body(t_ref, w_ref, o_ref):
    res = lax.dot_general(
        t_ref[...], w_ref[...],
        dimension_numbers=(((1,), (1,)), ((), ())),
        preferred_element_type=jnp.float32,
        precision=lax.Precision.HIGHEST,
    )
    o_ref[...] = jnp.concatenate([res, jnp.zeros_like(res)], axis=1)


def _time_proj(time_emb, W_time):
    return pl.pallas_call(
        _proj_body,
        out_shape=jax.ShapeDtypeStruct((NTAB, 2 * DIM), jnp.float32),
    )(time_emb, W_time)


def _add_body(so_es, so_eo, tp, es_o, eo_o):
    t = tp[...]
    es_o[...] = so_es[:, 0:DIM] + t
    eo_o[...] = so_eo[:, 0:DIM] + t


def _tproj_add(so128, tp2):
    blk = 1024
    out_t = jax.ShapeDtypeStruct((BATCH, DIM), jnp.float32)
    return pl.pallas_call(
        _add_body,
        grid=(BATCH // blk,),
        in_specs=[
            pl.BlockSpec((blk, 2 * DIM), lambda b: (b, 0)),
            pl.BlockSpec((blk, 2 * DIM), lambda b: (b + BATCH // blk, 0)),
            pl.BlockSpec((blk, DIM), lambda b: (b, 0)),
        ],
        out_specs=[
            pl.BlockSpec((blk, DIM), lambda b: (b, 0)),
            pl.BlockSpec((blk, DIM), lambda b: (b, 0)),
        ],
        out_shape=[out_t, out_t],
    )(so128, so128, tp2)


def _sc_body(subj_hbm, rel_idx_hbm, obj_hbm, time_idx_hbm,
             entT_hbm, rel128_hbm, proj128_hbm, tail128_hbm,
             er_out, tp_out, so_out,
             idx_r, idx_t, rows_t, rows_r, stage3,
             idxchunk, hitlist, slablist, superlist, tailbuf, slab, slabB,
             stage, stage_pos, tidx, tailrows,
             sem_t, sem_r, semA, semB):
    wid = lax.axis_index("s") * _NC + lax.axis_index("c")
    base = wid * _BPW
    bsl = pl.ds(base, _BPW)
    lanes = lax.iota(jnp.int32, _L)

    pltpu.sync_copy(rel_idx_hbm.at[bsl], idx_r)
    pltpu.sync_copy(time_idx_hbm.at[bsl], idx_t)
    for c in range(_NCHUNK):
        isl = pl.ds(c * _CH, _CH)
        gsl = pl.ds((base + c * _CH) // 8, _CH // 8)
        cp_t = pltpu.async_copy(proj128_hbm.at[idx_t.at[isl]], rows_t, sem_t)
        cp_r = pltpu.async_copy(rel128_hbm.at[idx_r.at[isl]], rows_r, sem_r)
        cp_r.wait()

        def pk_r(r, _):
            for j in range(DIM // _L):
                sl16 = pl.ds(j * _L, _L)
                stage3[r >> 3, r & 7, sl16] = rows_r[r, sl16]
            return 0

        lax.fori_loop(0, _CH, pk_r, 0)
        pltpu.sync_copy(stage3, er_out.at[gsl])
        cp_t.wait()

        def pk_t(r, _):
            for j in range(DIM // _L):
                sl16 = pl.ds(j * _L, _L)
                stage3[r >> 3, r & 7, sl16] = rows_t[r, sl16]
            return 0

        lax.fori_loop(0, _CH, pk_t, 0)
        pltpu.sync_copy(stage3, tp_out.at[gsl])

    base_e = wid * _RANGE
    lim_e = jnp.where(wid == _NW - 1, _TAIL0, base_e + _RANGE)
    hcount = jnp.int32(0)
    tcount = jnp.int32(0)
    for li, list_hbm in enumerate((subj_hbm, obj_hbm)):
        for c in range(BATCH // _ICH):
            pltpu.sync_copy(list_hbm.at[pl.ds(c * _ICH, _ICH)], idxchunk)

            def scan_vreg(j, hc, li=li, c=c):
                vs, ms, ss, pk = [], [], [], []
                for q in range(4):
                    v = idxchunk[pl.ds((4 * j + q) * _L, _L)]
                    posv = (li * BATCH + c * _ICH) + (4 * j + q) * _L + lanes
                    m = (v >= base_e) & (v < lim_e)
                    ms.append(m)
                    ss.append(jnp.sum(m.astype(jnp.int32)))
                    pk.append(((v - base_e) << 15) | posv)
                for q in range(4):
                    plsc.store_compressed(hitlist.at[pl.ds(hc, _L)], pk[q],
                                          mask=ms[q])
                    hc = hc + ss[q]
                return hc

            hcount = lax.fori_loop(0, _ICH // (4 * _L), scan_vreg, hcount)

    coff = pl.multiple_of(jnp.where(wid < 16, wid, wid - 16) * _ICH, 8)

    @pl.when(wid < 16)
    def _():
        pltpu.sync_copy(subj_hbm.at[pl.ds(coff, _ICH)], idxchunk)

    @pl.when(wid >= 16)
    def _():
        pltpu.sync_copy(obj_hbm.at[pl.ds(coff, _ICH)], idxchunk)

    def tail_scan(j, tc):
        v = idxchunk[pl.ds(j * _L, _L)]
        posv = wid * _ICH + j * _L + lanes
        mt = v >= _TAIL0
        packed_t = ((v - _TAIL0) << 16) | posv
        plsc.store_compressed(tailbuf.at[pl.ds(tc, _L)], packed_t, mask=mt)
        return tc + jnp.sum(mt.astype(jnp.int32))

    tcount = lax.fori_loop(0, _ICH // _L, tail_scan, tcount)
    for q in range(4):
        hitlist[pl.ds(hcount + q * _L, _L)] = jnp.full((_L,), 0x7F000000,
                                                       jnp.int32)
    tailbuf[pl.ds(tcount, _L)] = jnp.full((_L,), _DUMP, jnp.int32)

    nslabs = jnp.where(wid == _NW - 1, 62, 61)
    nv = (hcount + (2 * _L - 1)) >> 4

    def _load(s, buf, sem):
        s_c = jnp.minimum(s, nslabs - 1)
        e0 = pl.multiple_of(base_e + s_c * _SLAB, 128)
        pltpu.async_copy(entT_hbm.at[:, pl.ds(e0, _SLAB)], buf, sem)

    def _wait(buf, sem):
        pltpu.make_async_copy(entT_hbm.at[:, pl.ds(0, _SLAB)], buf, sem).wait()

    nv4 = (nv + 3) >> 2

    def _process(s, buf, nvs4, slot):
        def rv(j, k):
            ms, ss, vs = [], [], []
            for q in range(4):
                v = superlist[pl.ds((4 * j + q) * _L, _L)]
                m = (v >> 24) == s
                vs.append(v)
                ms.append(m)
                ss.append(jnp.sum(m.astype(jnp.int32)))
            for q in range(4):
                ko = jnp.minimum(k, _SCAP - _L)
                plsc.store_compressed(slablist.at[pl.ds(ko, _L)], vs[q],
                                      mask=ms[q])
                k = k + ss[q]
            return k

        K = jnp.minimum(lax.fori_loop(0, nvs4, rv, jnp.int32(0)), _SCAP - _L)

        def hit_body(h, slot):
            v16 = slablist[pl.ds((h >> 4) << 4, _L)]
            hv = jnp.sum(jnp.where(lanes == (h & 15), v16, 0))
            el = (hv >> 15) & (_SLAB - 1)
            pos = hv & 0x7FFF
            el_v = lax.broadcast(el, (_L,))
            for jj in range(DIM // _L):
                g = plsc.load_gather(buf, [lanes + jj * _L, el_v])
                stage[slot, pl.ds(jj * _L, _L)] = g
            stage_pos[...] = jnp.where(lanes == slot, pos, stage_pos[...])
            slot = slot + 1

            @pl.when(slot == _L)
            def _():
                pltpu.sync_copy(stage, so_out.at[stage_pos])

            return jnp.where(slot == _L, 0, slot)

        return lax.fori_loop(0, K, hit_body, slot)

    def super_body(u, slot):
        def sb(j, k):
            ms, ss, vs = [], [], []
            for q in range(4):
                v = hitlist[pl.ds((4 * j + q) * _L, _L)]
                m = (v >> 27) == u
                vs.append(v)
                ms.append(m)
                ss.append(jnp.sum(m.astype(jnp.int32)))
            for q in range(4):
                ko = jnp.minimum(k, _UCAP - _L)
                plsc.store_compressed(superlist.at[pl.ds(ko, _L)], vs[q],
                                      mask=ms[q])
                k = k + ss[q]
            return k

        ku = jnp.minimum(lax.fori_loop(0, nv4, sb, jnp.int32(0)), _UCAP - 4 * _L)
        for q in range(4):
            superlist[pl.ds(ku + q * _L, _L)] = jnp.full((_L,), 0x7F000000,
                                                         jnp.int32)
        nvs4 = (ku + 4 * _L + 63) >> 6

        def pair(i, slot):
            s0 = u * 8 + 2 * i
            _load(s0 + 1, slabB, semB)
            _wait(slab, semA)
            slot = _process(s0, slab, nvs4, slot)
            _load(s0 + 2, slab, semA)
            _wait(slabB, semB)
            slot = _process(s0 + 1, slabB, nvs4, slot)
            return slot

        return lax.fori_loop(0, 4, pair, slot)

    _load(0, slab, semA)
    slot = lax.fori_loop(0, 8, super_body, jnp.int32(0))
    _wait(slab, semA)
    stage_pos[...] = jnp.where(lanes < slot, stage_pos[...], _DUMP)

    @pl.when(slot > 0)
    def _():
        pltpu.sync_copy(stage, so_out.at[stage_pos])

    rounds = (tcount + _L - 1) >> 4

    def tail_round(t, _):
        v = tailbuf[pl.ds(t * _L, _L)]
        tidx[...] = (v >> 16) & 63
        pltpu.sync_copy(tail128_hbm.at[tidx], tailrows)
        stage_pos[...] = v & 0xFFFF
        pltpu.sync_copy(tailrows, so_out.at[stage_pos])
        return 0

    lax.fori_loop(0, rounds, tail_round, 0)


@jax.jit
def kernel(subjects, relations, objects, time_ids, ent_emb, rel_emb, time_emb,
           W_time):
    proj128 = _time_proj(time_emb, W_time)
    rel128 = jnp.pad(rel_emb, ((0, 0), (0, DIM)))
    tail128 = jnp.pad(lax.slice(ent_emb, (_TAIL0, 0), (NENT, DIM)),
                      ((0, 0), (0, DIM)))
    out3_t = jax.ShapeDtypeStruct((BATCH // 8, 8, DIM), jnp.float32)
    so_t = jax.ShapeDtypeStruct((_SO_ROWS, 2 * DIM), jnp.float32)
    mesh = plsc.VectorSubcoreMesh(core_axis_name="c", subcore_axis_name="s",
                                  num_cores=_NC, num_subcores=_NS)
    f = pl.kernel(
        _sc_body,
        out_type=[out3_t, out3_t, so_t],
        mesh=mesh,
        compiler_params=pltpu.CompilerParams(needs_layout_passes=False),
        scratch_types=[
            pltpu.VMEM((_BPW,), jnp.int32),
            pltpu.VMEM((_BPW,), jnp.int32),
            pltpu.VMEM((_CH, 2 * DIM), jnp.float32),
            pltpu.VMEM((_CH, 2 * DIM), jnp.float32),
            pltpu.VMEM((_CH // 8, 8, DIM), jnp.float32),
            pltpu.VMEM((_ICH,), jnp.int32),
            pltpu.VMEM((_HCAP,), jnp.int32),
            pltpu.VMEM((_SCAP,), jnp.int32),
            pltpu.VMEM((_UCAP,), jnp.int32),
            pltpu.VMEM((_ICH + _L,), jnp.int32),
            pltpu.VMEM((DIM, _SLAB), jnp.float32),
            pltpu.VMEM((DIM, _SLAB), jnp.float32),
            pltpu.VMEM((_L, 2 * DIM), jnp.float32),
            pltpu.VMEM((_L,), jnp.int32),
            pltpu.VMEM((_L,), jnp.int32),
            pltpu.VMEM((_L, 2 * DIM), jnp.float32),
            pltpu.SemaphoreType.DMA,
            pltpu.SemaphoreType.DMA,
            pltpu.SemaphoreType.DMA,
            pltpu.SemaphoreType.DMA,
        ],
    )
    er3, tp3, so128 = f(subjects, relations, objects, time_ids,
                        ent_emb.T, rel128, proj128, tail128)
    tp2 = tp3.reshape(BATCH, DIM)
    e_s, e_o = _tproj_add(so128, tp2)
    return (e_s, er3.reshape(BATCH, DIM), e_o, tp2)


# speedup vs baseline: 1.3912x; 1.3912x over previous
import functools

import jax
import jax.numpy as jnp
from jax import lax
from jax.experimental import pallas as pl
from jax.experimental.pallas import tpu as pltpu
from jax.experimental.pallas import tpu_sc as plsc

DIM = 64
BATCH = 16384
NENT = 1000000
NTAB = 1000
_L = 16
_NC = 2
_NS = 16
_NW = _NC * _NS
_BPW = BATCH // _NW
_CH = 32
_NCHUNK = _BPW // _CH

_RANGE = 31232
_SLAB = 512
_TAIL0 = _RANGE * 31 + 31744
_NTAIL = NENT - _TAIL0
_ICH = 1024
_HCAP = 2 * BATCH + 4 * _L
_SCAP = 4096
_UCAP = 4096 + 4 * _L
_SO_ROWS = 2 * BATCH + _L
_DUMP = 2 * BATCH


def _proj_body(t_ref, w_ref, o_ref):
    res = lax.dot_general(
        t_ref[...], w_ref[...],
        dimension_numbers=(((1,), (1,)), ((), ())),
        preferred_element_type=jnp.float32,
        precision=lax.Precision.HIGHEST,
    )
    o_ref[...] = jnp.concatenate([res, jnp.zeros_like(res)], axis=1)


def _time_proj(time_emb, W_time):
    return pl.pallas_call(
        _proj_body,
        out_shape=jax.ShapeDtypeStruct((NTAB, 2 * DIM), jnp.float32),
    )(time_emb, W_time)


def _add_body(so_es, so_eo, tp, es_o, eo_o):
    t = tp[...]
    es_o[...] = so_es[:, 0:DIM] + t
    eo_o[...] = so_eo[:, 0:DIM] + t


def _tproj_add(so128, tp2):
    blk = 1024
    out_t = jax.ShapeDtypeStruct((BATCH, DIM), jnp.float32)
    return pl.pallas_call(
        _add_body,
        grid=(BATCH // blk,),
        in_specs=[
            pl.BlockSpec((blk, 2 * DIM), lambda b: (b, 0)),
            pl.BlockSpec((blk, 2 * DIM), lambda b: (b + BATCH // blk, 0)),
            pl.BlockSpec((blk, DIM), lambda b: (b, 0)),
        ],
        out_specs=[
            pl.BlockSpec((blk, DIM), lambda b: (b, 0)),
            pl.BlockSpec((blk, DIM), lambda b: (b, 0)),
        ],
        out_shape=[out_t, out_t],
    )(so128, so128, tp2)


def _sc_body(subj_hbm, rel_idx_hbm, obj_hbm, time_idx_hbm,
             entT_hbm, rel128_hbm, proj128_hbm, tail128_hbm,
             er_out, tp_out, so_out,
             idx_r, idx_t, rows_t, rows_r, stage3,
             idxchunk, hitlist, slablist, tailbuf, slab, slabB,
             stage, stage_pos, tidx, tailrows,
             sem_t, sem_r, semA, semB):
    wid = lax.axis_index("s") * _NC + lax.axis_index("c")
    base = wid * _BPW
    bsl = pl.ds(base, _BPW)
    lanes = lax.iota(jnp.int32, _L)

    pltpu.sync_copy(rel_idx_hbm.at[bsl], idx_r)
    pltpu.sync_copy(time_idx_hbm.at[bsl], idx_t)
    for c in range(_NCHUNK):
        isl = pl.ds(c * _CH, _CH)
        gsl = pl.ds((base + c * _CH) // 8, _CH // 8)
        cp_t = pltpu.async_copy(proj128_hbm.at[idx_t.at[isl]], rows_t, sem_t)
        cp_r = pltpu.async_copy(rel128_hbm.at[idx_r.at[isl]], rows_r, sem_r)
        cp_r.wait()

        def pk_r(r, _):
            for j in range(DIM // _L):
                sl16 = pl.ds(j * _L, _L)
                stage3[r >> 3, r & 7, sl16] = rows_r[r, sl16]
            return 0

        lax.fori_loop(0, _CH, pk_r, 0)
        pltpu.sync_copy(stage3, er_out.at[gsl])
        cp_t.wait()

        def pk_t(r, _):
            for j in range(DIM // _L):
                sl16 = pl.ds(j * _L, _L)
                stage3[r >> 3, r & 7, sl16] = rows_t[r, sl16]
            return 0

        lax.fori_loop(0, _CH, pk_t, 0)
        pltpu.sync_copy(stage3, tp_out.at[gsl])

    base_e = wid * _RANGE
    lim_e = jnp.where(wid == _NW - 1, _TAIL0, base_e + _RANGE)
    hcount = jnp.int32(0)
    tcount = jnp.int32(0)
    for li, list_hbm in enumerate((subj_hbm, obj_hbm)):
        for c in range(BATCH // _ICH):
            pltpu.sync_copy(list_hbm.at[pl.ds(c * _ICH, _ICH)], idxchunk)

            def scan_vreg(j, hc, li=li, c=c):
                vs, ms, ss, pk = [], [], [], []
                for q in range(4):
                    v = idxchunk[pl.ds((4 * j + q) * _L, _L)]
                    posv = (li * BATCH + c * _ICH) + (4 * j + q) * _L + lanes
                    m = (v >= base_e) & (v < lim_e)
                    ms.append(m)
                    ss.append(jnp.sum(m.astype(jnp.int32)))
                    pk.append(((v - base_e) << 15) | posv)
                for q in range(4):
                    plsc.store_compressed(hitlist.at[pl.ds(hc, _L)], pk[q],
                                          mask=ms[q])
                    hc = hc + ss[q]
                return hc

            hcount = lax.fori_loop(0, _ICH // (4 * _L), scan_vreg, hcount)

    coff = pl.multiple_of(jnp.where(wid < 16, wid, wid - 16) * _ICH, 8)

    @pl.when(wid < 16)
    def _():
        pltpu.sync_copy(subj_hbm.at[pl.ds(coff, _ICH)], idxchunk)

    @pl.when(wid >= 16)
    def _():
        pltpu.sync_copy(obj_hbm.at[pl.ds(coff, _ICH)], idxchunk)

    def tail_scan(j, tc):
        v = idxchunk[pl.ds(j * _L, _L)]
        posv = wid * _ICH + j * _L + lanes
        mt = v >= _TAIL0
        packed_t = ((v - _TAIL0) << 16) | posv
        plsc.store_compressed(tailbuf.at[pl.ds(tc, _L)], packed_t, mask=mt)
        return tc + jnp.sum(mt.astype(jnp.int32))

    tcount = lax.fori_loop(0, _ICH // _L, tail_scan, tcount)
    for q in range(4):
        hitlist[pl.ds(hcount + q * _L, _L)] = jnp.full((_L,), 0x7F000000,
                                                       jnp.int32)
    tailbuf[pl.ds(tcount, _L)] = jnp.full((_L,), _DUMP, jnp.int32)

    nslabs = jnp.where(wid == _NW - 1, 62, 61)
    nv = (hcount + (2 * _L - 1)) >> 4

    def _load(s, buf, sem):
        s_c = jnp.minimum(s, nslabs - 1)
        e0 = pl.multiple_of(base_e + s_c * _SLAB, 128)
        pltpu.async_copy(entT_hbm.at[:, pl.ds(e0, _SLAB)], buf, sem)

    def _wait(buf, sem):
        pltpu.make_async_copy(entT_hbm.at[:, pl.ds(0, _SLAB)], buf, sem).wait()

    nv4 = (nv + 3) >> 2

    def _process(s, buf, slot):
        def rv(j, k):
            ms, ss, vs = [], [], []
            for q in range(4):
                v = hitlist[pl.ds((4 * j + q) * _L, _L)]
                m = (v >> 24) == s
                vs.append(v)
                ms.append(m)
                ss.append(jnp.sum(m.astype(jnp.int32)))
            for q in range(4):
                ko = jnp.minimum(k, _SCAP - _L)
                plsc.store_compressed(slablist.at[pl.ds(ko, _L)], vs[q],
                                      mask=ms[q])
                k = k + ss[q]
            return k

        K = jnp.minimum(lax.fori_loop(0, nv4, rv, jnp.int32(0)), _SCAP - _L)

        def hit_body(h, slot):
            v16 = slablist[pl.ds((h >> 4) << 4, _L)]
            hv = jnp.sum(jnp.where(lanes == (h & 15), v16, 0))
            el = (hv >> 15) & (_SLAB - 1)
            pos = hv & 0x7FFF
            el_v = lax.broadcast(el, (_L,))
            for jj in range(DIM // _L):
                g = plsc.load_gather(buf, [lanes + jj * _L, el_v])
                stage[slot, pl.ds(jj * _L, _L)] = g
            stage_pos[...] = jnp.where(lanes == slot, pos, stage_pos[...])
            slot = slot + 1

            @pl.when(slot == _L)
            def _():
                pltpu.sync_copy(stage, so_out.at[stage_pos])

            return jnp.where(slot == _L, 0, slot)

        return lax.fori_loop(0, K, hit_body, slot)

    _load(0, slab, semA)

    def pair(i, slot):
        s0 = 2 * i
        _load(s0 + 1, slabB, semB)
        _wait(slab, semA)
        slot = _process(s0, slab, slot)
        _load(s0 + 2, slab, semA)
        _wait(slabB, semB)
        slot = _process(s0 + 1, slabB, slot)
        return slot

    slot = lax.fori_loop(0, 31, pair, jnp.int32(0))
    _wait(slab, semA)
    stage_pos[...] = jnp.where(lanes < slot, stage_pos[...], _DUMP)

    @pl.when(slot > 0)
    def _():
        pltpu.sync_copy(stage, so_out.at[stage_pos])

    rounds = (tcount + _L - 1) >> 4

    def tail_round(t, _):
        v = tailbuf[pl.ds(t * _L, _L)]
        tidx[...] = (v >> 16) & 63
        pltpu.sync_copy(tail128_hbm.at[tidx], tailrows)
        stage_pos[...] = v & 0xFFFF
        pltpu.sync_copy(tailrows, so_out.at[stage_pos])
        return 0

    lax.fori_loop(0, rounds, tail_round, 0)


@jax.jit
def kernel(subjects, relations, objects, time_ids, ent_emb, rel_emb, time_emb,
           W_time):
    proj128 = _time_proj(time_emb, W_time)
    rel128 = jnp.pad(rel_emb, ((0, 0), (0, DIM)))
    tail128 = jnp.pad(lax.slice(ent_emb, (_TAIL0, 0), (NENT, DIM)),
                      ((0, 0), (0, DIM)))
    out3_t = jax.ShapeDtypeStruct((BATCH // 8, 8, DIM), jnp.float32)
    so_t = jax.ShapeDtypeStruct((_SO_ROWS, 2 * DIM), jnp.float32)
    mesh = plsc.VectorSubcoreMesh(core_axis_name="c", subcore_axis_name="s",
                                  num_cores=_NC, num_subcores=_NS)
    f = pl.kernel(
        _sc_body,
        out_type=[out3_t, out3_t, so_t],
        mesh=mesh,
        compiler_params=pltpu.CompilerParams(needs_layout_passes=False),
        scratch_types=[
            pltpu.VMEM((_BPW,), jnp.int32),
            pltpu.VMEM((_BPW,), jnp.int32),
            pltpu.VMEM((_CH, 2 * DIM), jnp.float32),
            pltpu.VMEM((_CH, 2 * DIM), jnp.float32),
            pltpu.VMEM((_CH // 8, 8, DIM), jnp.float32),
            pltpu.VMEM((_ICH,), jnp.int32),
            pltpu.VMEM((_HCAP,), jnp.int32),
            pltpu.VMEM((_SCAP,), jnp.int32),
            pltpu.VMEM((_ICH + _L,), jnp.int32),
            pltpu.VMEM((DIM, _SLAB), jnp.float32),
            pltpu.VMEM((DIM, _SLAB), jnp.float32),
            pltpu.VMEM((_L, 2 * DIM), jnp.float32),
            pltpu.VMEM((_L,), jnp.int32),
            pltpu.VMEM((_L,), jnp.int32),
            pltpu.VMEM((_L, 2 * DIM), jnp.float32),
            pltpu.SemaphoreType.DMA,
            pltpu.SemaphoreType.DMA,
            pltpu.SemaphoreType.DMA,
            pltpu.SemaphoreType.DMA,
        ],
    )
    er3, tp3, so128 = f(subjects, relations, objects, time_ids,
                        ent_emb.T, rel128, proj128, tail128)
    tp2 = tp3.reshape(BATCH, DIM)
    e_s, e_o = _tproj_add(so128, tp2)
    return (e_s, er3.reshape(BATCH, DIM), e_o, tp2)


# speedup vs baseline: 1.4659x; 1.0536x over previous
import functools

import jax
import jax.numpy as jnp
from jax import lax
from jax.experimental import pallas as pl
from jax.experimental.pallas import tpu as pltpu
from jax.experimental.pallas import tpu_sc as plsc

DIM = 64
BATCH = 16384
NENT = 1000000
NTAB = 1000
_L = 16
_NC = 2
_NS = 16
_NW = _NC * _NS
_BPW = BATCH // _NW
_CH = 32
_NCHUNK = _BPW // _CH

_RANGE = 31232
_SLAB = 512
_TAIL0 = _RANGE * 31 + 31744
_NTAIL = NENT - _TAIL0
_ICH = 1024
_HCAP = 2 * BATCH + 4 * _L
_SCAP = 4096
_UCAP = 4096 + 4 * _L
_SO_ROWS = 2 * BATCH + _L
_DUMP = 2 * BATCH


def _proj_body(t_ref, w_ref, o_ref):
    res = lax.dot_general(
        t_ref[...], w_ref[...],
        dimension_numbers=(((1,), (1,)), ((), ())),
        preferred_element_type=jnp.float32,
        precision=lax.Precision.HIGHEST,
    )
    o_ref[...] = jnp.concatenate([res, jnp.zeros_like(res)], axis=1)


def _time_proj(time_emb, W_time):
    return pl.pallas_call(
        _proj_body,
        out_shape=jax.ShapeDtypeStruct((NTAB, 2 * DIM), jnp.float32),
    )(time_emb, W_time)


def _add_body(so_es, so_eo, tp, es_o, eo_o):
    t = tp[...]
    es_o[...] = jnp.transpose(so_es[:, 0:DIM] + t)
    eo_o[...] = jnp.transpose(so_eo[:, 0:DIM] + t)


def _tproj_add(so128, tp2):
    blk = 1024
    out_t = jax.ShapeDtypeStruct((DIM, BATCH), jnp.float32)
    return pl.pallas_call(
        _add_body,
        grid=(BATCH // blk,),
        in_specs=[
            pl.BlockSpec((blk, 2 * DIM), lambda b: (b, 0)),
            pl.BlockSpec((blk, 2 * DIM), lambda b: (b + BATCH // blk, 0)),
            pl.BlockSpec((blk, DIM), lambda b: (b, 0)),
        ],
        out_specs=[
            pl.BlockSpec((DIM, blk), lambda b: (0, b)),
            pl.BlockSpec((DIM, blk), lambda b: (0, b)),
        ],
        out_shape=[out_t, out_t],
    )(so128, so128, tp2)


def _sc_body(subj_hbm, rel_idx_hbm, obj_hbm, time_idx_hbm,
             entT_hbm, rel128_hbm, proj128_hbm, tail128_hbm,
             er_out, tp_out, so_out,
             idx_r, idx_t, rows_t, rows_r, stage3,
             idxchunk, hitlist, slablist, tailbuf, slab, slabB,
             stage, stage_pos, tidx, tailrows,
             sem_t, sem_r, semA, semB):
    wid = lax.axis_index("s") * _NC + lax.axis_index("c")
    base = wid * _BPW
    bsl = pl.ds(base, _BPW)
    lanes = lax.iota(jnp.int32, _L)

    pltpu.sync_copy(rel_idx_hbm.at[bsl], idx_r)
    pltpu.sync_copy(time_idx_hbm.at[bsl], idx_t)
    for c in range(_NCHUNK):
        isl = pl.ds(c * _CH, _CH)
        gsl = pl.ds((base + c * _CH) // 8, _CH // 8)
        cp_t = pltpu.async_copy(proj128_hbm.at[idx_t.at[isl]], rows_t, sem_t)
        cp_r = pltpu.async_copy(rel128_hbm.at[idx_r.at[isl]], rows_r, sem_r)
        cp_r.wait()

        def pk_r(r, _):
            for j in range(DIM // _L):
                sl16 = pl.ds(j * _L, _L)
                stage3[r >> 3, r & 7, sl16] = rows_r[r, sl16]
            return 0

        lax.fori_loop(0, _CH, pk_r, 0)
        pltpu.sync_copy(stage3, er_out.at[gsl])
        cp_t.wait()

        def pk_t(r, _):
            for j in range(DIM // _L):
                sl16 = pl.ds(j * _L, _L)
                stage3[r >> 3, r & 7, sl16] = rows_t[r, sl16]
            return 0

        lax.fori_loop(0, _CH, pk_t, 0)
        pltpu.sync_copy(stage3, tp_out.at[gsl])

    base_e = wid * _RANGE
    lim_e = jnp.where(wid == _NW - 1, _TAIL0, base_e + _RANGE)
    hcount = jnp.int32(0)
    tcount = jnp.int32(0)
    for li, list_hbm in enumerate((subj_hbm, obj_hbm)):
        for c in range(BATCH // _ICH):
            pltpu.sync_copy(list_hbm.at[pl.ds(c * _ICH, _ICH)], idxchunk)

            def scan_vreg(j, hc, li=li, c=c):
                vs, ms, ss, pk = [], [], [], []
                for q in range(4):
                    v = idxchunk[pl.ds((4 * j + q) * _L, _L)]
                    posv = (li * BATCH + c * _ICH) + (4 * j + q) * _L + lanes
                    m = (v >= base_e) & (v < lim_e)
                    ms.append(m)
                    ss.append(jnp.sum(m.astype(jnp.int32)))
                    pk.append(((v - base_e) << 15) | posv)
                for q in range(4):
                    plsc.store_compressed(hitlist.at[pl.ds(hc, _L)], pk[q],
                                          mask=ms[q])
                    hc = hc + ss[q]
                return hc

            hcount = lax.fori_loop(0, _ICH // (4 * _L), scan_vreg, hcount)

    coff = pl.multiple_of(jnp.where(wid < 16, wid, wid - 16) * _ICH, 8)

    @pl.when(wid < 16)
    def _():
        pltpu.sync_copy(subj_hbm.at[pl.ds(coff, _ICH)], idxchunk)

    @pl.when(wid >= 16)
    def _():
        pltpu.sync_copy(obj_hbm.at[pl.ds(coff, _ICH)], idxchunk)

    def tail_scan(j, tc):
        v = idxchunk[pl.ds(j * _L, _L)]
        posv = wid * _ICH + j * _L + lanes
        mt = v >= _TAIL0
        packed_t = ((v - _TAIL0) << 16) | posv
        plsc.store_compressed(tailbuf.at[pl.ds(tc, _L)], packed_t, mask=mt)
        return tc + jnp.sum(mt.astype(jnp.int32))

    tcount = lax.fori_loop(0, _ICH // _L, tail_scan, tcount)
    for q in range(4):
        hitlist[pl.ds(hcount + q * _L, _L)] = jnp.full((_L,), 0x7F000000,
                                                       jnp.int32)
    tailbuf[pl.ds(tcount, _L)] = jnp.full((_L,), _DUMP, jnp.int32)

    nslabs = jnp.where(wid == _NW - 1, 62, 61)
    nv = (hcount + (2 * _L - 1)) >> 4

    def _load(s, buf, sem):
        s_c = jnp.minimum(s, nslabs - 1)
        e0 = pl.multiple_of(base_e + s_c * _SLAB, 128)
        pltpu.async_copy(entT_hbm.at[:, pl.ds(e0, _SLAB)], buf, sem)

    def _wait(buf, sem):
        pltpu.make_async_copy(entT_hbm.at[:, pl.ds(0, _SLAB)], buf, sem).wait()

    nv4 = (nv + 3) >> 2

    def _process(s, buf, slot):
        def rv(j, k):
            ms, ss, vs = [], [], []
            for q in range(4):
                v = hitlist[pl.ds((4 * j + q) * _L, _L)]
                m = (v >> 24) == s
                vs.append(v)
                ms.append(m)
                ss.append(jnp.sum(m.astype(jnp.int32)))
            for q in range(4):
                ko = jnp.minimum(k, _SCAP - _L)
                plsc.store_compressed(slablist.at[pl.ds(ko, _L)], vs[q],
                                      mask=ms[q])
                k = k + ss[q]
            return k

        K = jnp.minimum(lax.fori_loop(0, nv4, rv, jnp.int32(0)), _SCAP - _L)

        def hit_body(h, slot):
            v16 = slablist[pl.ds((h >> 4) << 4, _L)]
            hv = jnp.sum(jnp.where(lanes == (h & 15), v16, 0))
            el = (hv >> 15) & (_SLAB - 1)
            pos = hv & 0x7FFF
            el_v = lax.broadcast(el, (_L,))
            for jj in range(DIM // _L):
                g = plsc.load_gather(buf, [lanes + jj * _L, el_v])
                stage[slot, pl.ds(jj * _L, _L)] = g
            stage_pos[...] = jnp.where(lanes == slot, pos, stage_pos[...])
            slot = slot + 1

            @pl.when(slot == _L)
            def _():
                pltpu.sync_copy(stage, so_out.at[stage_pos])

            return jnp.where(slot == _L, 0, slot)

        return lax.fori_loop(0, K, hit_body, slot)

    _load(0, slab, semA)

    def pair(i, slot):
        s0 = 2 * i
        _load(s0 + 1, slabB, semB)
        _wait(slab, semA)
        slot = _process(s0, slab, slot)
        _load(s0 + 2, slab, semA)
        _wait(slabB, semB)
        slot = _process(s0 + 1, slabB, slot)
        return slot

    slot = lax.fori_loop(0, 31, pair, jnp.int32(0))
    _wait(slab, semA)
    stage_pos[...] = jnp.where(lanes < slot, stage_pos[...], _DUMP)

    @pl.when(slot > 0)
    def _():
        pltpu.sync_copy(stage, so_out.at[stage_pos])

    rounds = (tcount + _L - 1) >> 4

    def tail_round(t, _):
        v = tailbuf[pl.ds(t * _L, _L)]
        tidx[...] = (v >> 16) & 63
        pltpu.sync_copy(tail128_hbm.at[tidx], tailrows)
        stage_pos[...] = v & 0xFFFF
        pltpu.sync_copy(tailrows, so_out.at[stage_pos])
        return 0

    lax.fori_loop(0, rounds, tail_round, 0)


@jax.jit
def kernel(subjects, relations, objects, time_ids, ent_emb, rel_emb, time_emb,
           W_time):
    proj128 = _time_proj(time_emb, W_time)
    rel128 = jnp.pad(rel_emb, ((0, 0), (0, DIM)))
    tail128 = jnp.pad(lax.slice(ent_emb, (_TAIL0, 0), (NENT, DIM)),
                      ((0, 0), (0, DIM)))
    out3_t = jax.ShapeDtypeStruct((BATCH // 8, 8, DIM), jnp.float32)
    so_t = jax.ShapeDtypeStruct((_SO_ROWS, 2 * DIM), jnp.float32)
    mesh = plsc.VectorSubcoreMesh(core_axis_name="c", subcore_axis_name="s",
                                  num_cores=_NC, num_subcores=_NS)
    f = pl.kernel(
        _sc_body,
        out_type=[out3_t, out3_t, so_t],
        mesh=mesh,
        compiler_params=pltpu.CompilerParams(needs_layout_passes=False),
        scratch_types=[
            pltpu.VMEM((_BPW,), jnp.int32),
            pltpu.VMEM((_BPW,), jnp.int32),
            pltpu.VMEM((_CH, 2 * DIM), jnp.float32),
            pltpu.VMEM((_CH, 2 * DIM), jnp.float32),
            pltpu.VMEM((_CH // 8, 8, DIM), jnp.float32),
            pltpu.VMEM((_ICH,), jnp.int32),
            pltpu.VMEM((_HCAP,), jnp.int32),
            pltpu.VMEM((_SCAP,), jnp.int32),
            pltpu.VMEM((_ICH + _L,), jnp.int32),
            pltpu.VMEM((DIM, _SLAB), jnp.float32),
            pltpu.VMEM((DIM, _SLAB), jnp.float32),
            pltpu.VMEM((_L, 2 * DIM), jnp.float32),
            pltpu.VMEM((_L,), jnp.int32),
            pltpu.VMEM((_L,), jnp.int32),
            pltpu.VMEM((_L, 2 * DIM), jnp.float32),
            pltpu.SemaphoreType.DMA,
            pltpu.SemaphoreType.DMA,
            pltpu.SemaphoreType.DMA,
            pltpu.SemaphoreType.DMA,
        ],
    )
    er3, tp3, so128 = f(subjects, relations, objects, time_ids,
                        ent_emb.T, rel128, proj128, tail128)
    tp2 = tp3.reshape(BATCH, DIM)
    es_t, eo_t = _tproj_add(so128, tp2)
    return (es_t.T, er3.reshape(BATCH, DIM), eo_t.T, tp2)


# speedup vs baseline: 1.4735x; 1.0052x over previous
import jax
import jax.numpy as jnp
from jax import lax
from jax.experimental import pallas as pl
from jax.experimental.pallas import tpu as pltpu
from jax.experimental.pallas import tpu_sc as plsc

DIM = 64
BATCH = 16384
NENT = 1000000
NTAB = 1000
_L = 16
_NC = 2
_NS = 16
_NW = _NC * _NS
_BPW = BATCH // _NW
_CH = 32
_NCHUNK = _BPW // _CH

_RANGE = 31232
_SLAB = 512
_TAIL0 = _RANGE * 31 + 31744
_NTAIL = NENT - _TAIL0
_ICH = 1024
_HCAP = 2 * BATCH + 4 * _L
_SCAP = 4096
_SO_ROWS = 2 * BATCH + _L
_DUMP = 2 * BATCH


def _proj_body(t_ref, w_ref, o_ref):
    res = lax.dot_general(
        t_ref[...], w_ref[...],
        dimension_numbers=(((1,), (1,)), ((), ())),
        preferred_element_type=jnp.float32,
    )
    o_ref[...] = jnp.concatenate([res, jnp.zeros_like(res)], axis=1)


def _time_proj(time_emb, W_time):
    return pl.pallas_call(
        _proj_body,
        out_shape=jax.ShapeDtypeStruct((NTAB, 2 * DIM), jnp.float32),
    )(time_emb, W_time)


def _add_body(so_es, so_eo, tp, es_o, eo_o):
    t = tp[...]
    es_o[...] = jnp.transpose(so_es[:, 0:DIM] + t)
    eo_o[...] = jnp.transpose(so_eo[:, 0:DIM] + t)


def _tproj_add(so128, tp2):
    blk = 1024
    out_t = jax.ShapeDtypeStruct((DIM, BATCH), jnp.float32)
    return pl.pallas_call(
        _add_body,
        grid=(BATCH // blk,),
        in_specs=[
            pl.BlockSpec((blk, 2 * DIM), lambda b: (b, 0)),
            pl.BlockSpec((blk, 2 * DIM), lambda b: (b + BATCH // blk, 0)),
            pl.BlockSpec((blk, DIM), lambda b: (b, 0)),
        ],
        out_specs=[
            pl.BlockSpec((DIM, blk), lambda b: (0, b)),
            pl.BlockSpec((DIM, blk), lambda b: (0, b)),
        ],
        out_shape=[out_t, out_t],
    )(so128, so128, tp2)


def _sc_body(subj_hbm, rel_idx_hbm, obj_hbm, time_idx_hbm,
             entT_hbm, rel128_hbm, proj128_hbm, tail128_hbm,
             er_out, tp_out, so_out,
             idx_r, idx_t, rows_t, rows_r, stage3,
             idxchunk, hitlist, slablist, tailbuf, slab, slabB,
             stage, stage_pos, tidx, tailrows,
             sem_t, sem_r, semA, semB):
    wid = lax.axis_index("s") * _NC + lax.axis_index("c")
    base = wid * _BPW
    bsl = pl.ds(base, _BPW)
    lanes = lax.iota(jnp.int32, _L)

    pltpu.sync_copy(rel_idx_hbm.at[bsl], idx_r)
    pltpu.sync_copy(time_idx_hbm.at[bsl], idx_t)
    for c in range(_NCHUNK):
        isl = pl.ds(c * _CH, _CH)
        gsl = pl.ds((base + c * _CH) // 8, _CH // 8)
        cp_t = pltpu.async_copy(proj128_hbm.at[idx_t.at[isl]], rows_t, sem_t)
        cp_r = pltpu.async_copy(rel128_hbm.at[idx_r.at[isl]], rows_r, sem_r)
        cp_r.wait()

        def pk_r(r, _):
            for j in range(DIM // _L):
                sl16 = pl.ds(j * _L, _L)
                stage3[r >> 3, r & 7, sl16] = rows_r[r, sl16]
            return 0

        lax.fori_loop(0, _CH, pk_r, 0)
        pltpu.sync_copy(stage3, er_out.at[gsl])
        cp_t.wait()

        def pk_t(r, _):
            for j in range(DIM // _L):
                sl16 = pl.ds(j * _L, _L)
                stage3[r >> 3, r & 7, sl16] = rows_t[r, sl16]
            return 0

        lax.fori_loop(0, _CH, pk_t, 0)
        pltpu.sync_copy(stage3, tp_out.at[gsl])

    base_e = wid * _RANGE
    lim_e = jnp.where(wid == _NW - 1, _TAIL0, base_e + _RANGE)
    hcount = jnp.int32(0)
    tcount = jnp.int32(0)
    for li, list_hbm in enumerate((subj_hbm, obj_hbm)):
        for c in range(BATCH // _ICH):
            pltpu.sync_copy(list_hbm.at[pl.ds(c * _ICH, _ICH)], idxchunk)

            def scan_vreg(j, hc, li=li, c=c):
                vs, ms, ss, pk = [], [], [], []
                for q in range(4):
                    v = idxchunk[pl.ds((4 * j + q) * _L, _L)]
                    posv = (li * BATCH + c * _ICH) + (4 * j + q) * _L + lanes
                    m = (v >= base_e) & (v < lim_e)
                    ms.append(m)
                    ss.append(jnp.sum(m.astype(jnp.int32)))
                    pk.append(((v - base_e) << 15) | posv)
                for q in range(4):
                    plsc.store_compressed(hitlist.at[pl.ds(hc, _L)], pk[q],
                                          mask=ms[q])
                    hc = hc + ss[q]
                return hc

            hcount = lax.fori_loop(0, _ICH // (4 * _L), scan_vreg, hcount)

    coff = pl.multiple_of(jnp.where(wid < 16, wid, wid - 16) * _ICH, 8)

    @pl.when(wid < 16)
    def _():
        pltpu.sync_copy(subj_hbm.at[pl.ds(coff, _ICH)], idxchunk)

    @pl.when(wid >= 16)
    def _():
        pltpu.sync_copy(obj_hbm.at[pl.ds(coff, _ICH)], idxchunk)

    def tail_scan(j, tc):
        v = idxchunk[pl.ds(j * _L, _L)]
        posv = wid * _ICH + j * _L + lanes
        mt = v >= _TAIL0
        packed_t = ((v - _TAIL0) << 16) | posv
        plsc.store_compressed(tailbuf.at[pl.ds(tc, _L)], packed_t, mask=mt)
        return tc + jnp.sum(mt.astype(jnp.int32))

    tcount = lax.fori_loop(0, _ICH // _L, tail_scan, tcount)
    for q in range(4):
        hitlist[pl.ds(hcount + q * _L, _L)] = jnp.full((_L,), 0x7F000000,
                                                       jnp.int32)
    tailbuf[pl.ds(tcount, _L)] = jnp.full((_L,), _DUMP, jnp.int32)

    nslabs = jnp.where(wid == _NW - 1, 62, 61)
    nv = (hcount + (2 * _L - 1)) >> 4

    def _load(s, buf, sem):
        s_c = jnp.minimum(s, nslabs - 1)
        e0 = pl.multiple_of(base_e + s_c * _SLAB, 128)
        pltpu.async_copy(entT_hbm.at[:, pl.ds(e0, _SLAB)], buf, sem)

    def _wait(buf, sem):
        pltpu.make_async_copy(entT_hbm.at[:, pl.ds(0, _SLAB)], buf, sem).wait()

    nv4 = (nv + 3) >> 2

    def _process(s, buf, slot):
        def rv(j, k):
            ms, ss, vs = [], [], []
            for q in range(4):
                v = hitlist[pl.ds((4 * j + q) * _L, _L)]
                m = (v >> 24) == s
                vs.append(v)
                ms.append(m)
                ss.append(jnp.sum(m.astype(jnp.int32)))
            for q in range(4):
                ko = jnp.minimum(k, _SCAP - _L)
                plsc.store_compressed(slablist.at[pl.ds(ko, _L)], vs[q],
                                      mask=ms[q])
                k = k + ss[q]
            return k

        K = jnp.minimum(lax.fori_loop(0, nv4, rv, jnp.int32(0)), _SCAP - _L)

        def hit_body(h, slot):
            v16 = slablist[pl.ds((h >> 4) << 4, _L)]
            hv = jnp.sum(jnp.where(lanes == (h & 15), v16, 0))
            el = (hv >> 15) & (_SLAB - 1)
            pos = hv & 0x7FFF
            el_v = lax.broadcast(el, (_L,))
            for jj in range(DIM // _L):
                g = plsc.load_gather(buf, [lanes + jj * _L, el_v])
                stage[slot, pl.ds(jj * _L, _L)] = g
            stage_pos[...] = jnp.where(lanes == slot, pos, stage_pos[...])
            slot = slot + 1

            @pl.when(slot == _L)
            def _():
                pltpu.sync_copy(stage, so_out.at[stage_pos])

            return jnp.where(slot == _L, 0, slot)

        return lax.fori_loop(0, K, hit_body, slot)

    _load(0, slab, semA)

    def pair(i, slot):
        s0 = 2 * i
        _load(s0 + 1, slabB, semB)
        _wait(slab, semA)
        slot = _process(s0, slab, slot)
        _load(s0 + 2, slab, semA)
        _wait(slabB, semB)
        slot = _process(s0 + 1, slabB, slot)
        return slot

    slot = lax.fori_loop(0, 31, pair, jnp.int32(0))
    _wait(slab, semA)
    stage_pos[...] = jnp.where(lanes < slot, stage_pos[...], _DUMP)

    @pl.when(slot > 0)
    def _():
        pltpu.sync_copy(stage, so_out.at[stage_pos])

    rounds = (tcount + _L - 1) >> 4

    def tail_round(t, _):
        v = tailbuf[pl.ds(t * _L, _L)]
        tidx[...] = (v >> 16) & 63
        pltpu.sync_copy(tail128_hbm.at[tidx], tailrows)
        stage_pos[...] = v & 0xFFFF
        pltpu.sync_copy(tailrows, so_out.at[stage_pos])
        return 0

    lax.fori_loop(0, rounds, tail_round, 0)


@jax.jit
def kernel(subjects, relations, objects, time_ids, ent_emb, rel_emb, time_emb,
           W_time):
    proj128 = _time_proj(time_emb, W_time)
    rel128 = jnp.pad(rel_emb, ((0, 0), (0, DIM)))
    tail128 = jnp.pad(lax.slice(ent_emb, (_TAIL0, 0), (NENT, DIM)),
                      ((0, 0), (0, DIM)))
    out3_t = jax.ShapeDtypeStruct((BATCH // 8, 8, DIM), jnp.float32)
    so_t = jax.ShapeDtypeStruct((_SO_ROWS, 2 * DIM), jnp.float32)
    mesh = plsc.VectorSubcoreMesh(core_axis_name="c", subcore_axis_name="s",
                                  num_cores=_NC, num_subcores=_NS)
    f = pl.kernel(
        _sc_body,
        out_type=[out3_t, out3_t, so_t],
        mesh=mesh,
        compiler_params=pltpu.CompilerParams(needs_layout_passes=False),
        scratch_types=[
            pltpu.VMEM((_BPW,), jnp.int32),
            pltpu.VMEM((_BPW,), jnp.int32),
            pltpu.VMEM((_CH, 2 * DIM), jnp.float32),
            pltpu.VMEM((_CH, 2 * DIM), jnp.float32),
            pltpu.VMEM((_CH // 8, 8, DIM), jnp.float32),
            pltpu.VMEM((_ICH,), jnp.int32),
            pltpu.VMEM((_HCAP,), jnp.int32),
            pltpu.VMEM((_SCAP,), jnp.int32),
            pltpu.VMEM((_ICH + _L,), jnp.int32),
            pltpu.VMEM((DIM, _SLAB), jnp.float32),
            pltpu.VMEM((DIM, _SLAB), jnp.float32),
            pltpu.VMEM((_L, 2 * DIM), jnp.float32),
            pltpu.VMEM((_L,), jnp.int32),
            pltpu.VMEM((_L,), jnp.int32),
            pltpu.VMEM((_L, 2 * DIM), jnp.float32),
            pltpu.SemaphoreType.DMA,
            pltpu.SemaphoreType.DMA,
            pltpu.SemaphoreType.DMA,
            pltpu.SemaphoreType.DMA,
        ],
    )
    er3, tp3, so128 = f(subjects, relations, objects, time_ids,
                        ent_emb.T, rel128, proj128, tail128)
    tp2 = tp3.reshape(BATCH, DIM)
    es_t, eo_t = _tproj_add(so128, tp2)
    return (es_t.T, er3.reshape(BATCH, DIM), eo_t.T, tp2)
